# baseline (speedup 1.0000x reference)
import functools

import jax
import jax.numpy as jnp
from jax import lax
from jax.experimental import pallas as pl
from jax.experimental.pallas import tpu as pltpu

KEY_SIZE = 32
VALUE_SIZE = 256
QKV_SIZE = 2 * KEY_SIZE + VALUE_SIZE
TOPK = 128


def _ln(x, g, b, eps=1e-5):
    mu = x.mean(-1, keepdims=True)
    var = ((x - mu) ** 2).mean(-1, keepdims=True)
    return (x - mu) / jnp.sqrt(var + eps) * g + b


def _scores_body(c_ref, m_ref, o_ref):
    o_ref[...] = lax.dot_general(
        c_ref[...], m_ref[...], (((1,), (1,)), ((), ())))


def _scores(c, mem_c, interpret=False):
    B, C = c.shape
    N = mem_c.shape[0]
    BB, NB = 256, 4096
    return pl.pallas_call(
        _scores_body,
        grid=(B // BB, pl.cdiv(N, NB)),
        in_specs=[
            pl.BlockSpec((BB, C), lambda i, j: (i, 0)),
            pl.BlockSpec((NB, C), lambda i, j: (j, 0)),
        ],
        out_specs=pl.BlockSpec((BB, NB), lambda i, j: (i, j)),
        out_shape=jax.ShapeDtypeStruct((B, N), jnp.float32),
        interpret=interpret,
    )(c, mem_c)


def _dense_body(vals_ref, g_ref, Wq_ref, bq_ref, lng_ref, lnb_ref,
                lnmg_ref, lnmb_ref, W1_ref, b1_ref, W2_ref, b2_ref,
                Wp1_ref, bp1_ref, Wp2_ref, bp2_ref, o_ref):
    RB = vals_ref.shape[0]
    S = TOPK
    vals = vals_ref[...]
    G = g_ref[...]
    selr = lax.broadcasted_iota(jnp.int32, (RB * S, RB), 0) // S
    selc = lax.broadcasted_iota(jnp.int32, (RB * S, RB), 1)
    A_sel = (selr == selc).astype(jnp.float32)
    vals_b = jnp.dot(A_sel, vals)
    dj = lax.broadcasted_iota(jnp.int32, (RB * S, S), 0) % S
    ds = lax.broadcasted_iota(jnp.int32, (RB * S, S), 1)
    vflat = jnp.sum(vals_b * (dj == ds).astype(jnp.float32),
                    axis=1, keepdims=True)
    e0 = (lax.broadcasted_iota(jnp.int32, (1, VALUE_SIZE), 1) == 0)
    meta = G + vflat * e0.astype(jnp.float32)
    qkv = jnp.dot(meta, Wq_ref[...]) + bq_ref[...]
    qkv = _ln(qkv, lng_ref[...], lnb_ref[...])
    q = qkv[:, :KEY_SIZE] * (KEY_SIZE ** -0.5)
    kk = qkv[:, KEY_SIZE:2 * KEY_SIZE]
    v = qkv[:, 2 * KEY_SIZE:]
    att = []
    for r in range(RB):
        sl = slice(r * S, (r + 1) * S)
        dot = lax.dot_general(q[sl], kk[sl], (((1,), (1,)), ((), ())))
        w = jax.nn.softmax(dot, axis=-1)
        att.append(jnp.dot(w, v[sl]))
    att = jnp.concatenate(att, axis=0)
    m = _ln(meta + att, lnmg_ref[...], lnmb_ref[...])
    mlp = jnp.dot(jnp.maximum(jnp.dot(m, W1_ref[...]) + b1_ref[...], 0.0),
                  W2_ref[...]) + b2_ref[...]
    m2 = _ln(mlp + m, lnmg_ref[...], lnmb_ref[...])
    rows = lax.broadcasted_iota(jnp.int32, (RB, RB * S), 0)
    cols = lax.broadcasted_iota(jnp.int32, (RB, RB * S), 1)
    A = (cols // S == rows).astype(jnp.float32) * (1.0 / S)
    mmean = jnp.dot(A, m2)
    cp = jnp.dot(jnp.maximum(jnp.dot(mmean, Wp1_ref[...]) + bp1_ref[...], 0.0),
                 Wp2_ref[...]) + bp2_ref[...]
    o_ref[...] = cp


def _dense(vals, G, Wqkv, bqkv, ln_g, ln_b, lnm_g, lnm_b,
           W1, b1, W2, b2, Wp1, bp1, Wp2, bp2, interpret=False):
    B = vals.shape[0]
    RB = 8
    r2 = lambda x: x.reshape(1, -1)

    def full(a):
        nd = a.ndim
        return pl.BlockSpec(a.shape, lambda i, _nd=nd: (0,) * _nd)

    args = (vals, G, Wqkv, r2(bqkv), r2(ln_g), r2(ln_b), r2(lnm_g),
            r2(lnm_b), W1, r2(b1), W2, r2(b2), Wp1, r2(bp1), Wp2, r2(bp2))
    in_specs = [
        pl.BlockSpec((RB, TOPK), lambda i: (i, 0)),
        pl.BlockSpec((RB * TOPK, VALUE_SIZE), lambda i: (i, 0)),
    ] + [full(a) for a in args[2:]]
    return pl.pallas_call(
        _dense_body,
        grid=(B // RB,),
        in_specs=in_specs,
        out_specs=pl.BlockSpec((RB, VALUE_SIZE), lambda i: (i, 0)),
        out_shape=jax.ShapeDtypeStruct((B, VALUE_SIZE), jnp.float32),
        interpret=interpret,
    )(*args)


def kernel(c, k, mem_c, mem_t, mem_h, Wqkv, bqkv, ln_g, ln_b, lnm_g, lnm_b,
           W1, b1, W2, b2, Wp1, bp1, Wp2, bp2, interpret=False):
    del k
    N = mem_c.shape[0]
    deltas = _scores(c, mem_c, interpret=interpret)
    vals, idx = lax.top_k(deltas, TOPK)
    H_aug = jnp.concatenate(
        [jnp.zeros((N, 1), jnp.float32), mem_t, mem_h], axis=1)
    G = H_aug[idx.reshape(-1)]
    return _dense(vals, G, Wqkv, bqkv, ln_g, ln_b, lnm_g, lnm_b,
                  W1, b1, W2, b2, Wp1, bp1, Wp2, bp2, interpret=interpret)
